# baseline (speedup 1.0000x reference)
import functools

import jax
import jax.numpy as jnp
from jax import lax
from jax.experimental import pallas as pl
from jax.experimental.pallas import tpu as pltpu
from jax.experimental.pallas import tpu_sc as plsc

_TILE = 256


def _routing(sel, n_experts, tile, n_tiles):
    t_tok = sel.shape[0]
    order = jnp.argsort(sel, stable=True).astype(jnp.int32)
    sel_sorted = jnp.take(sel, order)
    counts = jnp.bincount(sel, length=n_experts).astype(jnp.int32)
    padded = ((counts + tile - 1) // tile) * tile
    ends_c = jnp.cumsum(counts)
    starts_c = ends_c - counts
    ends_p = jnp.cumsum(padded)
    starts_p = ends_p - padded
    rank = jnp.arange(t_tok, dtype=jnp.int32) - jnp.take(starts_c, sel_sorted)
    posj = jnp.take(starts_p, sel_sorted) + rank
    pt = n_tiles * tile
    src = jnp.zeros((pt,), jnp.int32).at[posj].set(order)
    pos = jnp.zeros((t_tok,), jnp.int32).at[order].set(posj)
    tile_start = jnp.arange(n_tiles, dtype=jnp.int32) * tile
    eids_raw = jnp.searchsorted(ends_p, tile_start, side="right").astype(jnp.int32)
    total = ends_p[-1]
    used = (tile_start < total).astype(jnp.int32)
    last_eid = jnp.take(eids_raw, total // tile - 1)
    eids = jnp.where(used == 1, eids_raw, last_eid)
    return src, pos, eids, used


def _sc_gather_rows(table, idx):
    _, d = table.shape
    b = idx.shape[0]
    info = plsc.get_sparse_core_info()
    nw = info.num_cores * info.num_subcores
    b_per_w = b // nw
    ch = b_per_w
    while ch > 128 or ch * d * 4 > 384 * 1024:
        ch //= 2
    nch = b_per_w // ch
    nbuf = min(2, nch)
    mesh = plsc.VectorSubcoreMesh(core_axis_name="c", subcore_axis_name="s")

    @functools.partial(
        pl.kernel,
        mesh=mesh,
        out_type=jax.ShapeDtypeStruct((b, d), table.dtype),
        scratch_types=(
            [pltpu.VMEM((b_per_w,), jnp.int32)]
            + [pltpu.VMEM((ch, d), table.dtype) for _ in range(nbuf)]
            + [pltpu.SemaphoreType.DMA for _ in range(nbuf)]
        ),
    )
    def gather_k(table_hbm, idx_hbm, out_hbm, idx_v, *bufs_sems):
        bufs = bufs_sems[:nbuf]
        sems = bufs_sems[nbuf:]
        wid = lax.axis_index("s") * info.num_cores + lax.axis_index("c")
        base = wid * b_per_w
        pltpu.sync_copy(idx_hbm.at[pl.ds(base, b_per_w)], idx_v)
        cps = [None] * nbuf
        for c in range(nch):
            cps[c % nbuf] = pltpu.async_copy(
                table_hbm.at[idx_v.at[pl.ds(c * ch, ch)]], bufs[c % nbuf],
                sems[c % nbuf])
            if c >= 1:
                j = (c - 1) % nbuf
                cps[j].wait()
                pltpu.sync_copy(bufs[j], out_hbm.at[pl.ds(base + (c - 1) * ch, ch)])
        j = (nch - 1) % nbuf
        cps[j].wait()
        pltpu.sync_copy(bufs[j], out_hbm.at[pl.ds(base + (nch - 1) * ch, ch)])

    return gather_k(table, idx)


def _swiglu_body(eids_ref, used_ref, x_ref, w_ref, b_ref, s_ref):
    t = pl.program_id(0)

    @pl.when(used_ref[t] == 1)
    def _():
        e = eids_ref[t]
        x = x_ref[...]
        w = w_ref[0].astype(jnp.bfloat16)
        h = jnp.dot(x, w, preferred_element_type=jnp.float32)
        h = h + b_ref[pl.ds(e, 1), :]
        half = h.shape[1] // 2
        a = h[:, :half]
        g = h[:, half:]
        s_ref[...] = ((a / (1.0 + jnp.exp(-a))) * g).astype(jnp.bfloat16)


def _mlp_body(eids_ref, used_ref, s_ref, w1_ref, b1_ref, w2_ref, b2_ref, o_ref,
              acc_ref):
    jc = pl.program_id(0)
    t = pl.program_id(1)
    nj = pl.num_programs(0)

    @pl.when(used_ref[t] == 1)
    def _():
        e = eids_ref[t]
        s = s_ref[...]
        tile = s.shape[0]
        w1 = w1_ref[0].astype(jnp.bfloat16)
        h1 = jnp.dot(s, w1, preferred_element_type=jnp.float32)
        h1 = jnp.maximum(h1 + b1_ref[pl.ds(e, 1), :], 0.0).astype(jnp.bfloat16)
        w2 = w2_ref[0].astype(jnp.bfloat16)
        part = jnp.dot(h1, w2, preferred_element_type=jnp.float32)
        sl = pl.ds(t * tile, tile)

        @pl.when(jc == 0)
        def _():
            acc_ref[sl, :] = part + b2_ref[pl.ds(e, 1), :]

        @pl.when(jc > 0)
        def _():
            acc_ref[sl, :] = acc_ref[sl, :] + part

        @pl.when(jc == nj - 1)
        def _():
            o_ref[...] = acc_ref[sl, :]


def kernel(x_parts, selected_experts, W12, b12, W1, b1, W2, b2):
    pp, nn, kk, f = x_parts.shape
    e_num, _, h2 = W12.shape
    h = h2 // 2
    out_d = W2.shape[-1]
    t_tok = pp * nn * kk
    tile = _TILE
    n_tiles = t_tok // tile + e_num
    pt = n_tiles * tile

    xf = x_parts.reshape(t_tok, f)
    sel = selected_experts.reshape(t_tok).astype(jnp.int32)
    src, pos, eids, used = _routing(sel, e_num, tile, n_tiles)

    x_pack = jax.lax.bitcast_convert_type(
        xf.astype(jnp.bfloat16).reshape(t_tok, f // 2, 2), jnp.int32)
    x_sorted = jax.lax.bitcast_convert_type(
        _sc_gather_rows(x_pack, src), jnp.bfloat16).reshape(pt, f)

    s = pl.pallas_call(
        _swiglu_body,
        grid_spec=pltpu.PrefetchScalarGridSpec(
            num_scalar_prefetch=2,
            grid=(n_tiles,),
            in_specs=[
                pl.BlockSpec((tile, f), lambda t, eids, used: (t, 0)),
                pl.BlockSpec((1, f, h2), lambda t, eids, used: (eids[t], 0, 0)),
                pl.BlockSpec((e_num, h2), lambda t, eids, used: (0, 0)),
            ],
            out_specs=pl.BlockSpec((tile, h), lambda t, eids, used: (t, 0)),
        ),
        out_shape=jax.ShapeDtypeStruct((pt, h), jnp.bfloat16),
        compiler_params=pltpu.CompilerParams(
            dimension_semantics=("arbitrary",),
        ),
    )(eids, used, x_sorted, W12, b12)

    hb = 768
    nj = h // hb
    out_sorted = pl.pallas_call(
        _mlp_body,
        grid_spec=pltpu.PrefetchScalarGridSpec(
            num_scalar_prefetch=2,
            grid=(nj, n_tiles),
            in_specs=[
                pl.BlockSpec((tile, h), lambda jc, t, eids, used: (t, 0)),
                pl.BlockSpec((1, h, hb), lambda jc, t, eids, used: (eids[t], 0, jc)),
                pl.BlockSpec((e_num, hb), lambda jc, t, eids, used: (0, jc)),
                pl.BlockSpec((1, hb, out_d), lambda jc, t, eids, used: (eids[t], jc, 0)),
                pl.BlockSpec((e_num, out_d), lambda jc, t, eids, used: (0, 0)),
            ],
            out_specs=pl.BlockSpec((tile, out_d), lambda jc, t, eids, used: (t, 0)),
            scratch_shapes=[pltpu.VMEM((pt, out_d), jnp.float32)],
        ),
        out_shape=jax.ShapeDtypeStruct((pt, out_d), jnp.float32),
        compiler_params=pltpu.CompilerParams(
            dimension_semantics=("arbitrary", "arbitrary"),
        ),
    )(eids, used, s, W1, b1, W2, b2)

    out_f = _sc_gather_rows(out_sorted, pos)
    return out_f.reshape(pp, nn, kk, out_d)


# speedup vs baseline: 2.6574x; 2.6574x over previous
import functools

import jax
import jax.numpy as jnp
from jax import lax
from jax.experimental import pallas as pl
from jax.experimental.pallas import tpu as pltpu
from jax.experimental.pallas import tpu_sc as plsc

_TILE = 256


def _routing(sel, n_experts, tile, n_tiles):
    t_tok = sel.shape[0]
    onehot = (sel[:, None] == jnp.arange(n_experts, dtype=jnp.int32)[None, :])
    cum = jnp.cumsum(onehot.astype(jnp.int32), axis=0)
    counts = cum[-1]
    rank = jnp.take_along_axis(cum, sel[:, None], axis=1)[:, 0] - 1
    padded = ((counts + tile - 1) // tile) * tile
    ends_p = jnp.cumsum(padded)
    starts_p = ends_p - padded
    pos = jnp.take(starts_p, sel) + rank
    pt = n_tiles * tile
    src = jnp.zeros((pt,), jnp.int32).at[pos].set(
        jnp.arange(t_tok, dtype=jnp.int32))
    tile_start = jnp.arange(n_tiles, dtype=jnp.int32) * tile
    eids_raw = jnp.searchsorted(ends_p, tile_start, side="right").astype(jnp.int32)
    total = ends_p[-1]
    used = (tile_start < total).astype(jnp.int32)
    last_eid = jnp.take(eids_raw, total // tile - 1)
    eids = jnp.where(used == 1, eids_raw, last_eid)
    return src, pos, eids, used


def _sc_gather_rows(table, idx):
    _, d = table.shape
    b = idx.shape[0]
    info = plsc.get_sparse_core_info()
    nw = info.num_cores * info.num_subcores
    b_per_w = b // nw
    ch = next(c for c in (128, 96, 64, 48, 32, 16, 8)
              if b_per_w % c == 0 and c * d * 4 <= 192 * 1024)
    nch = b_per_w // ch
    nbuf = min(2, nch)
    mesh = plsc.VectorSubcoreMesh(core_axis_name="c", subcore_axis_name="s")

    @functools.partial(
        pl.kernel,
        mesh=mesh,
        out_type=jax.ShapeDtypeStruct((b, d), table.dtype),
        scratch_types=(
            [pltpu.VMEM((b_per_w,), jnp.int32)]
            + [pltpu.VMEM((ch, d), table.dtype) for _ in range(nbuf)]
            + [pltpu.SemaphoreType.DMA for _ in range(nbuf)]
        ),
    )
    def gather_k(table_hbm, idx_hbm, out_hbm, idx_v, *bufs_sems):
        bufs = bufs_sems[:nbuf]
        sems = bufs_sems[nbuf:]
        wid = lax.axis_index("s") * info.num_cores + lax.axis_index("c")
        base = wid * b_per_w
        pltpu.sync_copy(idx_hbm.at[pl.ds(base, b_per_w)], idx_v)
        cps = [None] * nbuf
        for c in range(nch):
            cps[c % nbuf] = pltpu.async_copy(
                table_hbm.at[idx_v.at[pl.ds(c * ch, ch)]], bufs[c % nbuf],
                sems[c % nbuf])
            if c >= 1:
                j = (c - 1) % nbuf
                cps[j].wait()
                pltpu.sync_copy(bufs[j], out_hbm.at[pl.ds(base + (c - 1) * ch, ch)])
        j = (nch - 1) % nbuf
        cps[j].wait()
        pltpu.sync_copy(bufs[j], out_hbm.at[pl.ds(base + (nch - 1) * ch, ch)])

    return gather_k(table, idx)


def _swiglu_body(eids_ref, used_ref, x_ref, w_ref, b_ref, s_ref, wc_ref,
                 last_ref):
    t = pl.program_id(0)

    @pl.when(used_ref[t] == 1)
    def _():
        e = eids_ref[t]

        @pl.when((t == 0) | (e != last_ref[0]))
        def _():
            wc_ref[...] = w_ref[0].astype(jnp.bfloat16)
            last_ref[0] = e

        x = x_ref[...].astype(jnp.bfloat16)
        h = jnp.dot(x, wc_ref[...], preferred_element_type=jnp.float32)
        h = h + b_ref[pl.ds(e, 1), :]
        half = h.shape[1] // 2
        a = h[:, :half]
        g = h[:, half:]
        s_ref[...] = ((a / (1.0 + jnp.exp(-a))) * g).astype(jnp.bfloat16)


def _mlp_body(eids_ref, used_ref, s_ref, w1_ref, b1_ref, w2_ref, b2_ref, o_ref,
              acc_ref, w1c_ref, w2c_ref, last_ref):
    jc = pl.program_id(0)
    t = pl.program_id(1)
    nj = pl.num_programs(0)

    @pl.when(used_ref[t] == 1)
    def _():
        e = eids_ref[t]
        s = s_ref[...]
        tile = s.shape[0]

        @pl.when((t == 0) | (e != last_ref[0]))
        def _():
            w1c_ref[...] = w1_ref[0].astype(jnp.bfloat16)
            w2c_ref[...] = w2_ref[0].astype(jnp.bfloat16)
            last_ref[0] = e

        h1 = jnp.dot(s, w1c_ref[...], preferred_element_type=jnp.float32)
        h1 = jnp.maximum(h1 + b1_ref[pl.ds(e, 1), :], 0.0).astype(jnp.bfloat16)
        part = jnp.dot(h1, w2c_ref[...], preferred_element_type=jnp.float32)
        sl = pl.ds(t * tile, tile)

        @pl.when(jc == 0)
        def _():
            acc_ref[sl, :] = part + b2_ref[pl.ds(e, 1), :]

        @pl.when(jc > 0)
        def _():
            acc_ref[sl, :] = acc_ref[sl, :] + part

        @pl.when(jc == nj - 1)
        def _():
            o_ref[...] = acc_ref[sl, :]


def kernel(x_parts, selected_experts, W12, b12, W1, b1, W2, b2):
    pp, nn, kk, f = x_parts.shape
    e_num, _, h2 = W12.shape
    h = h2 // 2
    out_d = W2.shape[-1]
    t_tok = pp * nn * kk
    tile = _TILE
    n_tiles = t_tok // tile + e_num
    pt = n_tiles * tile

    xf = x_parts.reshape(t_tok, f)
    sel = selected_experts.reshape(t_tok).astype(jnp.int32)
    src, pos, eids, used = _routing(sel, e_num, tile, n_tiles)

    x_sorted = _sc_gather_rows(xf, src)

    s = pl.pallas_call(
        _swiglu_body,
        grid_spec=pltpu.PrefetchScalarGridSpec(
            num_scalar_prefetch=2,
            grid=(n_tiles,),
            in_specs=[
                pl.BlockSpec((tile, f), lambda t, eids, used: (t, 0)),
                pl.BlockSpec((1, f, h2), lambda t, eids, used: (eids[t], 0, 0)),
                pl.BlockSpec((e_num, h2), lambda t, eids, used: (0, 0)),
            ],
            out_specs=pl.BlockSpec((tile, h), lambda t, eids, used: (t, 0)),
            scratch_shapes=[pltpu.VMEM((f, h2), jnp.bfloat16),
                            pltpu.SMEM((1,), jnp.int32)],
        ),
        out_shape=jax.ShapeDtypeStruct((pt, h), jnp.bfloat16),
        compiler_params=pltpu.CompilerParams(
            dimension_semantics=("arbitrary",),
        ),
    )(eids, used, x_sorted, W12, b12)

    hb = 768
    nj = h // hb
    out_sorted = pl.pallas_call(
        _mlp_body,
        grid_spec=pltpu.PrefetchScalarGridSpec(
            num_scalar_prefetch=2,
            grid=(nj, n_tiles),
            in_specs=[
                pl.BlockSpec((tile, h), lambda jc, t, eids, used: (t, 0)),
                pl.BlockSpec((1, h, hb), lambda jc, t, eids, used: (eids[t], 0, jc)),
                pl.BlockSpec((e_num, hb), lambda jc, t, eids, used: (0, jc)),
                pl.BlockSpec((1, hb, out_d), lambda jc, t, eids, used: (eids[t], jc, 0)),
                pl.BlockSpec((e_num, out_d), lambda jc, t, eids, used: (0, 0)),
            ],
            out_specs=pl.BlockSpec(
                (tile, out_d),
                lambda jc, t, eids, used: (jnp.where(jc == nj - 1, t, 0), 0)),
            scratch_shapes=[pltpu.VMEM((pt, out_d), jnp.float32),
                            pltpu.VMEM((h, hb), jnp.bfloat16),
                            pltpu.VMEM((hb, out_d), jnp.bfloat16),
                            pltpu.SMEM((1,), jnp.int32)],
        ),
        out_shape=jax.ShapeDtypeStruct((pt, out_d), jnp.float32),
        compiler_params=pltpu.CompilerParams(
            dimension_semantics=("arbitrary", "arbitrary"),
        ),
    )(eids, used, s, W1, b1, W2, b2)

    out_f = _sc_gather_rows(out_sorted, pos)
    return out_f.reshape(pp, nn, kk, out_d)


# speedup vs baseline: 3.2087x; 1.2075x over previous
import functools

import jax
import jax.numpy as jnp
from jax import lax
from jax.experimental import pallas as pl
from jax.experimental.pallas import tpu as pltpu
from jax.experimental.pallas import tpu_sc as plsc

_TILE = 256


def _routing(sel, n_experts, tile, n_tiles):
    t_tok = sel.shape[0]
    onehot = (sel[:, None] == jnp.arange(n_experts, dtype=jnp.int32)[None, :])
    cum = jnp.cumsum(onehot.astype(jnp.int32), axis=0)
    counts = cum[-1]
    rank = jnp.take_along_axis(cum, sel[:, None], axis=1)[:, 0] - 1
    padded = ((counts + tile - 1) // tile) * tile
    ends_p = jnp.cumsum(padded)
    starts_p = ends_p - padded
    pos = jnp.take(starts_p, sel) + rank
    tile_start = jnp.arange(n_tiles, dtype=jnp.int32) * tile
    eids_raw = jnp.searchsorted(ends_p, tile_start, side="right").astype(jnp.int32)
    total = ends_p[-1]
    used = (tile_start < total).astype(jnp.int32)
    last_eid = jnp.take(eids_raw, total // tile - 1)
    eids = jnp.where(used == 1, eids_raw, last_eid)
    return pos, eids, used


def _sc_gather_rows(table, idx):
    _, d = table.shape
    b = idx.shape[0]
    info = plsc.get_sparse_core_info()
    nw = info.num_cores * info.num_subcores
    b_per_w = b // nw
    ch = next(c for c in (128, 96, 64, 48, 32, 16, 8)
              if b_per_w % c == 0 and c * d * 4 <= 192 * 1024)
    nch = b_per_w // ch
    nbuf = min(2, nch)
    mesh = plsc.VectorSubcoreMesh(core_axis_name="c", subcore_axis_name="s")

    @functools.partial(
        pl.kernel,
        mesh=mesh,
        out_type=jax.ShapeDtypeStruct((b, d), table.dtype),
        scratch_types=(
            [pltpu.VMEM((b_per_w,), jnp.int32)]
            + [pltpu.VMEM((ch, d), table.dtype) for _ in range(nbuf)]
            + [pltpu.SemaphoreType.DMA for _ in range(nbuf)]
        ),
    )
    def gather_k(table_hbm, idx_hbm, out_hbm, idx_v, *bufs_sems):
        bufs = bufs_sems[:nbuf]
        sems = bufs_sems[nbuf:]
        wid = lax.axis_index("s") * info.num_cores + lax.axis_index("c")
        base = wid * b_per_w
        pltpu.sync_copy(idx_hbm.at[pl.ds(base, b_per_w)], idx_v)
        cps = [None] * nbuf
        for c in range(nch):
            cps[c % nbuf] = pltpu.async_copy(
                table_hbm.at[idx_v.at[pl.ds(c * ch, ch)]], bufs[c % nbuf],
                sems[c % nbuf])
            if c >= 1:
                j = (c - 1) % nbuf
                cps[j].wait()
                pltpu.sync_copy(bufs[j], out_hbm.at[pl.ds(base + (c - 1) * ch, ch)])
        j = (nch - 1) % nbuf
        cps[j].wait()
        pltpu.sync_copy(bufs[j], out_hbm.at[pl.ds(base + (nch - 1) * ch, ch)])

    return gather_k(table, idx)


def _sc_scatter_rows(rows, idx3, out_rows):
    n, d = rows.shape
    nw_, nch, ch = idx3.shape
    mesh = plsc.VectorSubcoreMesh(core_axis_name="c", subcore_axis_name="s")
    info = plsc.get_sparse_core_info()
    b_per_w = n // nw_

    @functools.partial(
        pl.kernel,
        mesh=mesh,
        out_type=jax.ShapeDtypeStruct((out_rows, d), rows.dtype),
        scratch_types=(
            [pltpu.VMEM((nch, ch), jnp.int32)]
            + [pltpu.VMEM((ch, d), rows.dtype) for _ in range(nch)]
            + [pltpu.SemaphoreType.DMA for _ in range(nch)]
        ),
    )
    def scatter_k(rows_hbm, idx_hbm, out_hbm, idx_v, *bufs_sems):
        bufs = bufs_sems[:nch]
        sems = bufs_sems[nch:]
        wid = lax.axis_index("s") * info.num_cores + lax.axis_index("c")
        base = wid * b_per_w
        pltpu.sync_copy(idx_hbm.at[wid], idx_v)
        cps = []
        for c in range(nch):
            pltpu.sync_copy(rows_hbm.at[pl.ds(base + c * ch, ch)], bufs[c])
            cps.append(pltpu.async_copy(bufs[c], out_hbm.at[idx_v.at[c]],
                                        sems[c]))
        for cp in cps:
            cp.wait()

    return scatter_k(rows, idx3)


def _swiglu_body(eids_ref, used_ref, x_ref, w_ref, b_ref, s_ref, wc_ref,
                 last_ref):
    t = pl.program_id(0)

    @pl.when(used_ref[t] == 1)
    def _():
        e = eids_ref[t]

        @pl.when((t == 0) | (e != last_ref[0]))
        def _():
            wc_ref[...] = w_ref[0].astype(jnp.bfloat16)
            last_ref[0] = e

        x = x_ref[...].astype(jnp.bfloat16)
        h = jnp.dot(x, wc_ref[...], preferred_element_type=jnp.float32)
        h = h + b_ref[pl.ds(e, 1), :]
        half = h.shape[1] // 2
        a = h[:, :half]
        g = h[:, half:]
        s_ref[...] = ((a / (1.0 + jnp.exp(-a))) * g).astype(jnp.bfloat16)


def _mlp_body(eids_ref, used_ref, s_ref, w1_ref, b1_ref, w2_ref, b2_ref, o_ref,
              acc_ref, w1c_ref, w2c_ref, last_ref):
    jc = pl.program_id(0)
    t = pl.program_id(1)
    nj = pl.num_programs(0)

    @pl.when(used_ref[t] == 1)
    def _():
        e = eids_ref[t]
        s = s_ref[...]
        tile = s.shape[0]

        @pl.when((t == 0) | (e != last_ref[0]))
        def _():
            w1c_ref[...] = w1_ref[0].astype(jnp.bfloat16)
            w2c_ref[...] = w2_ref[0].astype(jnp.bfloat16)
            last_ref[0] = e

        h1 = jnp.dot(s, w1c_ref[...], preferred_element_type=jnp.float32)
        h1 = jnp.maximum(h1 + b1_ref[pl.ds(e, 1), :], 0.0).astype(jnp.bfloat16)
        part = jnp.dot(h1, w2c_ref[...], preferred_element_type=jnp.float32)
        sl = pl.ds(t * tile, tile)

        @pl.when(jc == 0)
        def _():
            acc_ref[sl, :] = part + b2_ref[pl.ds(e, 1), :]

        @pl.when(jc > 0)
        def _():
            acc_ref[sl, :] = acc_ref[sl, :] + part

        @pl.when(jc == nj - 1)
        def _():
            o_ref[...] = acc_ref[sl, :]


def kernel(x_parts, selected_experts, W12, b12, W1, b1, W2, b2):
    pp, nn, kk, f = x_parts.shape
    e_num, _, h2 = W12.shape
    h = h2 // 2
    out_d = W2.shape[-1]
    t_tok = pp * nn * kk
    tile = _TILE
    n_tiles = t_tok // tile + e_num
    pt = n_tiles * tile

    xf = x_parts.reshape(t_tok, f)
    sel = selected_experts.reshape(t_tok).astype(jnp.int32)
    pos, eids, used = _routing(sel, e_num, tile, n_tiles)

    info = plsc.get_sparse_core_info()
    nw = info.num_cores * info.num_subcores
    ch = 64
    pos3 = pos.reshape(nw, t_tok // (nw * ch), ch)
    x_sorted = _sc_scatter_rows(xf, pos3, pt)

    s = pl.pallas_call(
        _swiglu_body,
        grid_spec=pltpu.PrefetchScalarGridSpec(
            num_scalar_prefetch=2,
            grid=(n_tiles,),
            in_specs=[
                pl.BlockSpec((tile, f), lambda t, eids, used: (t, 0)),
                pl.BlockSpec((1, f, h2), lambda t, eids, used: (eids[t], 0, 0)),
                pl.BlockSpec((e_num, h2), lambda t, eids, used: (0, 0)),
            ],
            out_specs=pl.BlockSpec((tile, h), lambda t, eids, used: (t, 0)),
            scratch_shapes=[pltpu.VMEM((f, h2), jnp.bfloat16),
                            pltpu.SMEM((1,), jnp.int32)],
        ),
        out_shape=jax.ShapeDtypeStruct((pt, h), jnp.bfloat16),
        compiler_params=pltpu.CompilerParams(
            dimension_semantics=("arbitrary",),
        ),
    )(eids, used, x_sorted, W12, b12)

    hb = 768
    nj = h // hb
    out_sorted = pl.pallas_call(
        _mlp_body,
        grid_spec=pltpu.PrefetchScalarGridSpec(
            num_scalar_prefetch=2,
            grid=(nj, n_tiles),
            in_specs=[
                pl.BlockSpec((tile, h), lambda jc, t, eids, used: (t, 0)),
                pl.BlockSpec((1, h, hb), lambda jc, t, eids, used: (eids[t], 0, jc)),
                pl.BlockSpec((e_num, hb), lambda jc, t, eids, used: (0, jc)),
                pl.BlockSpec((1, hb, out_d), lambda jc, t, eids, used: (eids[t], jc, 0)),
                pl.BlockSpec((e_num, out_d), lambda jc, t, eids, used: (0, 0)),
            ],
            out_specs=pl.BlockSpec(
                (tile, out_d),
                lambda jc, t, eids, used: (jnp.where(jc == nj - 1, t, 0), 0)),
            scratch_shapes=[pltpu.VMEM((pt, out_d), jnp.float32),
                            pltpu.VMEM((h, hb), jnp.bfloat16),
                            pltpu.VMEM((hb, out_d), jnp.bfloat16),
                            pltpu.SMEM((1,), jnp.int32)],
        ),
        out_shape=jax.ShapeDtypeStruct((pt, out_d), jnp.float32),
        compiler_params=pltpu.CompilerParams(
            dimension_semantics=("arbitrary", "arbitrary"),
        ),
    )(eids, used, s, W1, b1, W2, b2)

    out_f = _sc_gather_rows(out_sorted, pos)
    return out_f.reshape(pp, nn, kk, out_d)


# speedup vs baseline: 3.2404x; 1.0099x over previous
import functools

import jax
import jax.numpy as jnp
from jax import lax
from jax.experimental import pallas as pl
from jax.experimental.pallas import tpu as pltpu
from jax.experimental.pallas import tpu_sc as plsc

_TILE = 256


def _routing(sel, n_experts, tile, n_tiles):
    t_tok = sel.shape[0]
    onehot = (sel[:, None] == jnp.arange(n_experts, dtype=jnp.int32)[None, :])
    cum = jnp.cumsum(onehot.astype(jnp.int32), axis=0)
    counts = cum[-1]
    padded = ((counts + tile - 1) // tile) * tile
    ends_p = jnp.cumsum(padded)
    starts_p = ends_p - padded
    pos = jnp.sum(jnp.where(onehot, starts_p[None, :] + cum - 1, 0), axis=1)
    tile_start = jnp.arange(n_tiles, dtype=jnp.int32) * tile
    eids_raw = jnp.sum(
        (ends_p[None, :] <= tile_start[:, None]).astype(jnp.int32), axis=1)
    total = ends_p[-1]
    used = (tile_start < total).astype(jnp.int32)
    lu = total // tile - 1
    last_eid = jnp.sum(
        jnp.where(jnp.arange(n_tiles, dtype=jnp.int32) == lu, eids_raw, 0))
    eids = jnp.where(used == 1, eids_raw, last_eid)
    return pos.astype(jnp.int32), eids, used


def _sc_gather_rows(table, idx):
    _, d = table.shape
    b = idx.shape[0]
    info = plsc.get_sparse_core_info()
    nw = info.num_cores * info.num_subcores
    b_per_w = b // nw
    ch = next(c for c in (128, 96, 64, 48, 32, 16, 8)
              if b_per_w % c == 0 and c * d * 4 <= 192 * 1024)
    nch = b_per_w // ch
    nbuf = min(2, nch)
    mesh = plsc.VectorSubcoreMesh(core_axis_name="c", subcore_axis_name="s")

    @functools.partial(
        pl.kernel,
        mesh=mesh,
        out_type=jax.ShapeDtypeStruct((b, d), table.dtype),
        scratch_types=(
            [pltpu.VMEM((b_per_w,), jnp.int32)]
            + [pltpu.VMEM((ch, d), table.dtype) for _ in range(nbuf)]
            + [pltpu.SemaphoreType.DMA for _ in range(nbuf)]
        ),
    )
    def gather_k(table_hbm, idx_hbm, out_hbm, idx_v, *bufs_sems):
        bufs = bufs_sems[:nbuf]
        sems = bufs_sems[nbuf:]
        wid = lax.axis_index("s") * info.num_cores + lax.axis_index("c")
        base = wid * b_per_w
        pltpu.sync_copy(idx_hbm.at[pl.ds(base, b_per_w)], idx_v)
        cps = [None] * nbuf
        for c in range(nch):
            cps[c % nbuf] = pltpu.async_copy(
                table_hbm.at[idx_v.at[pl.ds(c * ch, ch)]], bufs[c % nbuf],
                sems[c % nbuf])
            if c >= 1:
                j = (c - 1) % nbuf
                cps[j].wait()
                pltpu.sync_copy(bufs[j], out_hbm.at[pl.ds(base + (c - 1) * ch, ch)])
        j = (nch - 1) % nbuf
        cps[j].wait()
        pltpu.sync_copy(bufs[j], out_hbm.at[pl.ds(base + (nch - 1) * ch, ch)])

    return gather_k(table, idx)


def _sc_scatter_rows(rows, idx3, out_rows):
    n, d = rows.shape
    nw_, nch, ch = idx3.shape
    mesh = plsc.VectorSubcoreMesh(core_axis_name="c", subcore_axis_name="s")
    info = plsc.get_sparse_core_info()
    b_per_w = n // nw_

    @functools.partial(
        pl.kernel,
        mesh=mesh,
        out_type=jax.ShapeDtypeStruct((out_rows, d), rows.dtype),
        scratch_types=(
            [pltpu.VMEM((nch, ch), jnp.int32)]
            + [pltpu.VMEM((ch, d), rows.dtype) for _ in range(nch)]
            + [pltpu.SemaphoreType.DMA for _ in range(nch)]
        ),
    )
    def scatter_k(rows_hbm, idx_hbm, out_hbm, idx_v, *bufs_sems):
        bufs = bufs_sems[:nch]
        sems = bufs_sems[nch:]
        wid = lax.axis_index("s") * info.num_cores + lax.axis_index("c")
        base = wid * b_per_w
        pltpu.sync_copy(idx_hbm.at[wid], idx_v)
        cps = []
        for c in range(nch):
            pltpu.sync_copy(rows_hbm.at[pl.ds(base + c * ch, ch)], bufs[c])
            cps.append(pltpu.async_copy(bufs[c], out_hbm.at[idx_v.at[c]],
                                        sems[c]))
        for cp in cps:
            cp.wait()

    return scatter_k(rows, idx3)


def _swiglu_body(eids_ref, used_ref, x_ref, wa0_ref, wa1_ref, wg0_ref,
                 wg1_ref, b_ref, s_ref, ca0_ref, ca1_ref, cg0_ref, cg1_ref,
                 last_ref):
    t = pl.program_id(0)

    @pl.when(used_ref[t] == 1)
    def _():
        e = eids_ref[t]

        @pl.when((t == 0) | (e != last_ref[0]))
        def _():
            ca0_ref[...] = wa0_ref[0].astype(jnp.bfloat16)
            ca1_ref[...] = wa1_ref[0].astype(jnp.bfloat16)
            cg0_ref[...] = wg0_ref[0].astype(jnp.bfloat16)
            cg1_ref[...] = wg1_ref[0].astype(jnp.bfloat16)
            last_ref[0] = e

        x = x_ref[...].astype(jnp.bfloat16)
        q = ca0_ref.shape[1]
        half = 2 * q
        for i, (ca, cg) in enumerate(((ca0_ref, cg0_ref), (ca1_ref, cg1_ref))):
            a = jnp.dot(x, ca[...], preferred_element_type=jnp.float32)
            a = a + b_ref[pl.ds(e, 1), pl.ds(i * q, q)]
            g = jnp.dot(x, cg[...], preferred_element_type=jnp.float32)
            g = g + b_ref[pl.ds(e, 1), pl.ds(half + i * q, q)]
            s_ref[:, pl.ds(i * q, q)] = (
                (a / (1.0 + jnp.exp(-a))) * g).astype(jnp.bfloat16)


def _mlp_body(eids_ref, used_ref, s_ref, w10_ref, w11_ref, w12_ref, w13_ref,
              b1_ref, w2_ref, b2_ref, o_ref, acc_ref, c0_ref, c1_ref, c2_ref,
              c3_ref, w2c_ref, last_ref):
    jc = pl.program_id(0)
    t = pl.program_id(1)
    nj = pl.num_programs(0)
    w1_refs = (w10_ref, w11_ref, w12_ref, w13_ref)
    c_refs = (c0_ref, c1_ref, c2_ref, c3_ref)

    @pl.when(used_ref[t] == 1)
    def _():
        e = eids_ref[t]
        s = s_ref[...]
        tile = s.shape[0]

        @pl.when((t == 0) | (e != last_ref[0]))
        def _():
            for wr, cr in zip(w1_refs, c_refs):
                cr[...] = wr[0].astype(jnp.bfloat16)
            w2c_ref[...] = w2_ref[0].astype(jnp.bfloat16)
            last_ref[0] = e

        q = c0_ref.shape[0]
        h1 = jnp.dot(s[:, :q], c0_ref[...], preferred_element_type=jnp.float32)
        for i in range(1, 4):
            h1 = h1 + jnp.dot(s[:, i * q:(i + 1) * q], c_refs[i][...],
                              preferred_element_type=jnp.float32)
        h1 = jnp.maximum(h1 + b1_ref[pl.ds(e, 1), :], 0.0).astype(jnp.bfloat16)
        part = jnp.dot(h1, w2c_ref[...], preferred_element_type=jnp.float32)
        sl = pl.ds(t * tile, tile)

        @pl.when(jc == 0)
        def _():
            acc_ref[sl, :] = part + b2_ref[pl.ds(e, 1), :]

        @pl.when(jc > 0)
        def _():
            acc_ref[sl, :] = acc_ref[sl, :] + part

        @pl.when(jc == nj - 1)
        def _():
            o_ref[...] = acc_ref[sl, :]


def kernel(x_parts, selected_experts, W12, b12, W1, b1, W2, b2):
    pp, nn, kk, f = x_parts.shape
    e_num, _, h2 = W12.shape
    h = h2 // 2
    out_d = W2.shape[-1]
    t_tok = pp * nn * kk
    tile = _TILE
    n_tiles = t_tok // tile + e_num
    pt = n_tiles * tile

    xf = x_parts.reshape(t_tok, f)
    sel = selected_experts.reshape(t_tok).astype(jnp.int32)
    pos, eids, used = _routing(sel, e_num, tile, n_tiles)

    info = plsc.get_sparse_core_info()
    nw = info.num_cores * info.num_subcores
    ch = 64
    pos3 = pos.reshape(nw, t_tok // (nw * ch), ch)
    x_sorted = _sc_scatter_rows(xf, pos3, pt)

    q1 = h2 // 4
    s = pl.pallas_call(
        _swiglu_body,
        grid_spec=pltpu.PrefetchScalarGridSpec(
            num_scalar_prefetch=2,
            grid=(n_tiles,),
            in_specs=[
                pl.BlockSpec((tile, f), lambda t, eids, used: (t, 0)),
            ] + [
                pl.BlockSpec((1, f, q1),
                             lambda t, eids, used, j=j: (eids[t], 0, j))
                for j in range(4)
            ] + [
                pl.BlockSpec((e_num, h2), lambda t, eids, used: (0, 0)),
            ],
            out_specs=pl.BlockSpec((tile, h), lambda t, eids, used: (t, 0)),
            scratch_shapes=[pltpu.VMEM((f, q1), jnp.bfloat16)
                            for _ in range(4)]
                           + [pltpu.SMEM((1,), jnp.int32)],
        ),
        out_shape=jax.ShapeDtypeStruct((pt, h), jnp.bfloat16),
        compiler_params=pltpu.CompilerParams(
            dimension_semantics=("arbitrary",),
        ),
    )(eids, used, x_sorted, W12, W12, W12, W12, b12)

    hb = 768
    nj = h // hb
    out_sorted = pl.pallas_call(
        _mlp_body,
        grid_spec=pltpu.PrefetchScalarGridSpec(
            num_scalar_prefetch=2,
            grid=(nj, n_tiles),
            in_specs=[
                pl.BlockSpec((tile, h), lambda jc, t, eids, used: (t, 0)),
            ] + [
                pl.BlockSpec((1, h // 4, hb),
                             lambda jc, t, eids, used, r=r: (eids[t], r, jc))
                for r in range(4)
            ] + [
                pl.BlockSpec((e_num, hb), lambda jc, t, eids, used: (0, jc)),
                pl.BlockSpec((1, hb, out_d),
                             lambda jc, t, eids, used: (eids[t], jc, 0)),
                pl.BlockSpec((e_num, out_d), lambda jc, t, eids, used: (0, 0)),
            ],
            out_specs=pl.BlockSpec(
                (tile, out_d),
                lambda jc, t, eids, used: (jnp.where(jc == nj - 1, t, 0), 0)),
            scratch_shapes=[pltpu.VMEM((pt, out_d), jnp.float32)]
                           + [pltpu.VMEM((h // 4, hb), jnp.bfloat16)
                              for _ in range(4)]
                           + [pltpu.VMEM((hb, out_d), jnp.bfloat16),
                              pltpu.SMEM((1,), jnp.int32)],
        ),
        out_shape=jax.ShapeDtypeStruct((pt, out_d), jnp.float32),
        compiler_params=pltpu.CompilerParams(
            dimension_semantics=("arbitrary", "arbitrary"),
        ),
    )(eids, used, s, W1, W1, W1, W1, b1, W2, b2)

    out_f = _sc_gather_rows(out_sorted, pos)
    return out_f.reshape(pp, nn, kk, out_d)
